# baseline (speedup 1.0000x reference)
import functools

import jax
import jax.numpy as jnp
from jax import lax
from jax.experimental import pallas as pl
from jax.experimental.pallas import tpu as pltpu
from jax.experimental.pallas import tpu_sc as plsc

NC = 2
NS = 16
NW = NC * NS

NBUF = 8
TNB = 4
TCW = 128


@functools.lru_cache(maxsize=None)
def _make_detile(vocab, dim):
    n_full = vocab // TCW
    tail = vocab - n_full * TCW
    per_tile = -(-n_full // NW)
    n_groups = -(-per_tile // TNB)

    mesh = plsc.VectorSubcoreMesh(core_axis_name="c", subcore_axis_name="s")

    @functools.partial(
        pl.kernel,
        mesh=mesh,
        compiler_params=pltpu.CompilerParams(needs_layout_passes=False),
        out_type=jax.ShapeDtypeStruct((vocab // 2, 2 * dim), jnp.float32),
        scratch_types=[
            pltpu.VMEM((TNB, dim, TCW), jnp.float32),
            pltpu.VMEM((TNB, TCW // 2, 2 * dim), jnp.float32),
            pltpu.VMEM((dim, 64), jnp.float32),
            pltpu.SemaphoreType.DMA((TNB,)),
            pltpu.SemaphoreType.DMA((TNB,)),
        ],
    )
    def detile(wt_hbm, out_hbm, vin, vout, tvin, isem, osem):
        wid = lax.axis_index("s") * NC + lax.axis_index("c")

        rows0 = lax.iota(jnp.int32, 16)
        row_vecs = [rows0 + 16 * s for s in range(dim // 16)]

        def chunk_of(i):
            return wid + NW * i

        def copy_in(i, b):
            return pltpu.make_async_copy(
                wt_hbm.at[:, pl.ds(chunk_of(i) * TCW, TCW)],
                vin.at[b], isem.at[b])

        def copy_out(i, b):
            return pltpu.make_async_copy(
                vout.at[b],
                out_hbm.at[pl.ds(chunk_of(i) * (TCW // 2), TCW // 2)],
                osem.at[b])

        UNROLL = 8

        def transpose_rows(src, b, n_rows):
            def jl_body(g, _):
                jl0 = g * UNROLL
                for u in range(UNROLL):
                    jl = jl0 + u
                    for h in range(2):
                        col = jnp.full((16,), 2 * jl + h, jnp.int32)
                        for s in range(dim // 16):
                            vout[b, jl, pl.ds(h * dim + 16 * s, 16)] = (
                                plsc.load_gather(src, [row_vecs[s], col]))
                return 0

            lax.fori_loop(0, n_rows // UNROLL, jl_body, 0)

        for b in range(TNB):

            @pl.when(chunk_of(b) < n_full)
            def _():
                copy_in(b, b).start()

        def group(g, _):
            i0 = g * TNB
            for b in range(TNB):
                i = i0 + b

                @pl.when(chunk_of(i) < n_full)
                def _():
                    copy_in(i, b).wait()

                    @pl.when(i >= TNB)
                    def _():
                        copy_out(i - TNB, b).wait()

                    transpose_rows(vin.at[b], b, TCW // 2)
                    copy_out(i, b).start()

                    @pl.when(chunk_of(i + TNB) < n_full)
                    def _():
                        copy_in(i + TNB, b).start()
            return 0

        lax.fori_loop(0, n_groups, group, 0)

        for j in range((n_groups - 2) * TNB, n_groups * TNB):

            @pl.when(
                (chunk_of(j) < n_full) & (chunk_of(j + TNB) >= n_full))
            def _():
                copy_out(j, j % TNB).wait()

        if tail:
            @pl.when(wid == 0)
            def _():
                pltpu.make_async_copy(
                    wt_hbm.at[:, pl.ds(n_full * TCW, tail)],
                    tvin, isem.at[0]).start()
                pltpu.make_async_copy(
                    wt_hbm.at[:, pl.ds(n_full * TCW, tail)],
                    tvin, isem.at[0]).wait()
                transpose_rows(tvin, 0, tail // 2)
                pltpu.make_async_copy(
                    vout.at[0, pl.ds(0, tail // 2)],
                    out_hbm.at[pl.ds(n_full * (TCW // 2), tail // 2)],
                    osem.at[0]).start()
                pltpu.make_async_copy(
                    vout.at[0, pl.ds(0, tail // 2)],
                    out_hbm.at[pl.ds(n_full * (TCW // 2), tail // 2)],
                    osem.at[0]).wait()

    return detile


@functools.lru_cache(maxsize=None)
def _make_lookup(batch, k, vocab, dim):
    rows_per_w = batch // NW
    assert rows_per_w % NBUF == 0

    mesh = plsc.VectorSubcoreMesh(core_axis_name="c", subcore_axis_name="s")

    @functools.partial(
        pl.kernel,
        mesh=mesh,
        compiler_params=pltpu.CompilerParams(use_tc_tiling_on_sc=False),
        out_type=jax.ShapeDtypeStruct((batch, 24, 128), jnp.float32),
        scratch_types=[
            pltpu.VMEM((rows_per_w, k), jnp.int32),
            pltpu.VMEM((NBUF, k, dim), jnp.float32),
            pltpu.SemaphoreType.DMA((NBUF,)),
            pltpu.SemaphoreType.DMA((NBUF,)),
        ],
    )
    def lookup(x_hbm, w_hbm, out_hbm, idx_v, rows_v, gsem, ssem):
        wid = lax.axis_index("s") * NC + lax.axis_index("c")
        row0 = wid * rows_per_w

        pltpu.sync_copy(x_hbm.at[pl.ds(row0, rows_per_w)], idx_v)

        def gather(r, b):
            return pltpu.make_async_copy(
                w_hbm.at[idx_v.at[r]], rows_v.at[b], gsem.at[b])

        def scatter(r, b):
            return pltpu.make_async_copy(
                rows_v.at[b],
                out_hbm.at[row0 + r, pl.ds(0, k), pl.ds(0, dim)],
                ssem.at[b])

        for b in range(NBUF):
            gather(b, b).start()

        def group(i, _):
            i0 = i * NBUF
            for b in range(NBUF):
                r = i0 + b
                gather(r, b).wait()
                scatter(r, b).start()
                f = r + NBUF

                @pl.when(f < rows_per_w)
                def _():
                    scatter(r, b).wait()
                    gather(f, b).start()
            return 0

        lax.fori_loop(0, rows_per_w // NBUF, group, 0)

        for b in range(NBUF):
            scatter(rows_per_w - NBUF + b, b).wait()

    return lookup


def kernel(x, weight):
    batch, k = x.shape
    vocab, dim = weight.shape
    wlin = _make_detile(vocab, dim)(weight.T).reshape(vocab, dim)
    out = _make_lookup(batch, k, vocab, dim)(x.astype(jnp.int32), wlin)
    return out[:, :k, :dim]


# speedup vs baseline: 2.4037x; 2.4037x over previous
import functools

import jax
import jax.numpy as jnp
from jax import lax
from jax.experimental import pallas as pl
from jax.experimental.pallas import tpu as pltpu
from jax.experimental.pallas import tpu_sc as plsc

NC = 2
NS = 16
NW = NC * NS

NBUF = 8


@functools.lru_cache(maxsize=None)
def _make_lookup(batch, k, vocab2, dim):
    rows_per_w = batch // NW
    assert rows_per_w % NBUF == 0

    mesh = plsc.VectorSubcoreMesh(core_axis_name="c", subcore_axis_name="s")

    @functools.partial(
        pl.kernel,
        mesh=mesh,
        compiler_params=pltpu.CompilerParams(use_tc_tiling_on_sc=False),
        out_type=jax.ShapeDtypeStruct((batch, 24, 128), jnp.float32),
        scratch_types=[
            pltpu.VMEM((rows_per_w, k), jnp.int32),
            pltpu.VMEM((NBUF, k, dim), jnp.float32),
            pltpu.SemaphoreType.DMA((NBUF,)),
            pltpu.SemaphoreType.DMA((NBUF,)),
        ],
    )
    def lookup(x_hbm, w_hbm, out_hbm, idx_v, rows_v, gsem, ssem):
        wid = lax.axis_index("s") * NC + lax.axis_index("c")
        row0 = wid * rows_per_w

        pltpu.sync_copy(x_hbm.at[pl.ds(row0, rows_per_w)], idx_v)

        def gather(r, b):
            return pltpu.make_async_copy(
                w_hbm.at[idx_v.at[r]], rows_v.at[b], gsem.at[b])

        def scatter(r, b):
            return pltpu.make_async_copy(
                rows_v.at[b],
                out_hbm.at[row0 + r, pl.ds(0, k), pl.ds(0, dim)],
                ssem.at[b])

        for b in range(NBUF):
            gather(b, b).start()

        def group(i, _):
            i0 = i * NBUF
            for b in range(NBUF):
                r = i0 + b
                gather(r, b).wait()
                scatter(r, b).start()
                f = r + NBUF

                @pl.when(f < rows_per_w)
                def _():
                    scatter(r, b).wait()
                    gather(f, b).start()
            return 0

        lax.fori_loop(0, rows_per_w // NBUF, group, 0)

        for b in range(NBUF):
            scatter(rows_per_w - NBUF + b, b).wait()

    return lookup


def kernel(x, weight):
    batch, k = x.shape
    vocab, dim = weight.shape
    wp = jnp.pad(weight, ((0, 0), (0, dim))).reshape(2 * vocab, dim)
    x2 = x.astype(jnp.int32) * 2
    out = _make_lookup(batch, k, 2 * vocab, dim)(x2, wp)
    return out[:, :k, :dim]
